# baseline (speedup 1.0000x reference)
import jax
import jax.numpy as jnp
from jax import lax
from jax.experimental import pallas as pl
from jax.experimental.pallas import tpu as pltpu
from jax.experimental.pallas import tpu_sc as plsc

NEG_SLOPE = 0.05

N = 10000
D = 128
E = 320000
V = 8000

NC = 2
NS = 16
NW = NC * NS

CH = 128
K = 80
NBUF = 2
WCH = 16
T = K * CH
EPAD = NW * T
RPT = 632
NROW = NS * RPT

VPAD = 8192
GCH = 128
VK = VPAD // (NW * GCH)

_mesh = plsc.VectorSubcoreMesh(core_axis_name="c", subcore_axis_name="s")


def _scatter_body(x_hbm, col_hbm, row_hbm, zeros_hbm, out_hbm,
                  colv, rowv, rows_a, rows_b, acc, sem_g):
    cid = lax.axis_index("c")
    sid = lax.axis_index("s")
    wid = sid * NC + cid

    pltpu.sync_copy(zeros_hbm.at[pl.ds(sid * RPT, RPT)],
                    acc.at[pl.ds(sid * RPT, RPT)])
    plsc.subcore_barrier()

    def win(w, carry):
        pltpu.sync_copy(col_hbm.at[wid, pl.ds(w * WCH, WCH)], colv)
        pltpu.sync_copy(row_hbm.at[wid, pl.ds(w * WCH, WCH)], rowv)
        pltpu.async_copy(x_hbm.at[colv.at[0]], rows_a, sem_g)

        def group(g, carry2):
            j = 2 * g
            pltpu.make_async_copy(x_hbm.at[pl.ds(0, CH)], rows_a,
                                  sem_g).wait()
            pltpu.async_copy(x_hbm.at[colv.at[j + 1]], rows_b, sem_g)
            pltpu.sync_copy(rows_a, acc.at[rowv.at[j]], add=True)
            pltpu.make_async_copy(x_hbm.at[pl.ds(0, CH)], rows_b,
                                  sem_g).wait()
            nxt = jnp.minimum(j + 2, WCH - 1)
            pltpu.async_copy(x_hbm.at[colv.at[nxt]], rows_a, sem_g)
            pltpu.sync_copy(rows_b, acc.at[rowv.at[j + 1]], add=True)
            return carry2

        lax.fori_loop(0, WCH // 2, group, 0)
        pltpu.make_async_copy(x_hbm.at[pl.ds(0, CH)], rows_a,
                              sem_g).wait()
        return carry

    lax.fori_loop(0, K // WCH, win, 0)
    plsc.subcore_barrier()

    pltpu.sync_copy(acc.at[pl.ds(sid * RPT, RPT)],
                    out_hbm.at[cid, pl.ds(sid * RPT, RPT)])


_scatter_k = pl.kernel(
    _scatter_body,
    mesh=_mesh,
    out_type=jax.ShapeDtypeStruct((NC, NROW, D), jnp.float32),
    scratch_types=[
        pltpu.VMEM((WCH, CH), jnp.int32),
        pltpu.VMEM((WCH, CH), jnp.int32),
        pltpu.VMEM((CH, D), jnp.float32),
        pltpu.VMEM((CH, D), jnp.float32),
        pltpu.VMEM_SHARED((NROW, D), jnp.float32),
        pltpu.SemaphoreType.DMA,
    ],
)


def _combine_body(p0_ref, p1_ref, x_ref, b_ref, o_ref):
    s = p0_ref[...] + p1_ref[...] + x_ref[...] + b_ref[...]
    o_ref[...] = jnp.maximum(s, NEG_SLOPE * s)


def _combine(p0, p1, x, b2):
    blk = 400
    return pl.pallas_call(
        _combine_body,
        grid=(N // blk,),
        in_specs=[
            pl.BlockSpec((blk, D), lambda i: (i, 0)),
            pl.BlockSpec((blk, D), lambda i: (i, 0)),
            pl.BlockSpec((blk, D), lambda i: (i, 0)),
            pl.BlockSpec((1, D), lambda i: (0, 0)),
        ],
        out_specs=pl.BlockSpec((blk, D), lambda i: (i, 0)),
        out_shape=jax.ShapeDtypeStruct((N, D), jnp.float32),
    )(p0, p1, x, b2)


def _gather_body(h_hbm, idx_hbm, out_hbm, idxv, rows_v, sem):
    cid = lax.axis_index("c")
    sid = lax.axis_index("s")
    wid = sid * NC + cid
    pltpu.sync_copy(idx_hbm.at[wid], idxv)

    def chunk(j, carry):
        pltpu.async_copy(h_hbm.at[idxv.at[j]], rows_v, sem).wait()
        pltpu.sync_copy(rows_v,
                        out_hbm.at[pl.ds(wid * VK * GCH + j * GCH, GCH)])
        return carry

    lax.fori_loop(0, VK, chunk, 0)


_gather_k = pl.kernel(
    _gather_body,
    mesh=_mesh,
    out_type=jax.ShapeDtypeStruct((VPAD, D), jnp.float32),
    scratch_types=[
        pltpu.VMEM((VK, GCH), jnp.int32),
        pltpu.VMEM((GCH, D), jnp.float32),
        pltpu.SemaphoreType.DMA,
    ],
)


def _prep_edges(edges):
    pad = EPAD - E
    col = jnp.concatenate([edges[1], jnp.zeros((pad,), jnp.int32)])
    row = jnp.concatenate([edges[0], jnp.full((pad,), N, jnp.int32)])
    return col.reshape(NW, K, CH), row.reshape(NW, K, CH)


def kernel(embedding, bias, edges1, edges2, idx_mapping):
    col1, row1 = _prep_edges(edges1)
    col2, row2 = _prep_edges(edges2)
    zeros = jnp.zeros((NROW, D), jnp.float32)
    b2 = bias.reshape(1, D)

    p = _scatter_k(embedding, col1, row1, zeros)
    h1 = _combine(p[0], p[1], embedding, b2)
    p2 = _scatter_k(h1, col2, row2, zeros)
    h2 = _combine(p2[0], p2[1], h1, b2)

    idx = jnp.concatenate([idx_mapping, jnp.zeros((VPAD - V,), jnp.int32)])
    out = _gather_k(h2, idx.reshape(NW, VK, GCH))
    return out[:V]


# speedup vs baseline: 1.7347x; 1.7347x over previous
import functools

import jax
import jax.numpy as jnp
from jax import lax
from jax.experimental import pallas as pl
from jax.experimental.pallas import tpu as pltpu
from jax.experimental.pallas import tpu_sc as plsc

NEG_SLOPE = 0.05

N = 10000
D = 128
E = 320000
V = 8000

NC = 2
NS = 16
NW = NC * NS

CH = 128
K0 = 95
K1 = 62
KMAX = K0
RPT = 632
NROW = NS * RPT

VPAD = 8192
GCH = 128
VK = VPAD // (NW * GCH)

_mesh = plsc.VectorSubcoreMesh(core_axis_name="c", subcore_axis_name="s")


def _scatter_body(x_hbm, col_hbm, row_hbm, zeros_hbm, out_hbm,
                  colv, rowv, rows_a, acc, sem_g):
    cid = lax.axis_index("c")
    sid = lax.axis_index("s")
    wid = cid * NS + sid

    pltpu.sync_copy(zeros_hbm.at[pl.ds(sid * RPT, RPT)],
                    acc.at[pl.ds(sid * RPT, RPT)])
    plsc.subcore_barrier()

    pltpu.sync_copy(col_hbm.at[wid], colv)
    pltpu.sync_copy(row_hbm.at[wid], rowv)

    def group(g, carry):
        pltpu.async_copy(x_hbm.at[colv.at[g]], rows_a, sem_g).wait()
        pltpu.sync_copy(rows_a, acc.at[rowv.at[g]], add=True)
        return carry

    kb = jnp.where(cid == 0, K0, K1)
    lax.fori_loop(0, kb, group, 0)
    plsc.subcore_barrier()

    pltpu.sync_copy(acc.at[pl.ds(sid * RPT, RPT)],
                    out_hbm.at[cid, pl.ds(sid * RPT, RPT)])


_scatter_k = pl.kernel(
    _scatter_body,
    mesh=_mesh,
    out_type=jax.ShapeDtypeStruct((NC, NROW, D), jnp.float32),
    scratch_types=[
        pltpu.VMEM((KMAX, CH), jnp.int32),
        pltpu.VMEM((KMAX, CH), jnp.int32),
        pltpu.VMEM((CH, D), jnp.float32),
        pltpu.VMEM_SHARED((NROW, D), jnp.float32),
        pltpu.SemaphoreType.DMA,
    ],
)


def _combine_body(p0_ref, p1_ref, x_ref, b_ref, o_ref):
    s = p0_ref[...] + p1_ref[...] + x_ref[...] + b_ref[...]
    o_ref[...] = jnp.maximum(s, NEG_SLOPE * s)


def _combine(p0, p1, x, b2):
    blk = 400
    return pl.pallas_call(
        _combine_body,
        grid=(N // blk,),
        in_specs=[
            pl.BlockSpec((blk, D), lambda i: (i, 0)),
            pl.BlockSpec((blk, D), lambda i: (i, 0)),
            pl.BlockSpec((blk, D), lambda i: (i, 0)),
            pl.BlockSpec((1, D), lambda i: (0, 0)),
        ],
        out_specs=pl.BlockSpec((blk, D), lambda i: (i, 0)),
        out_shape=jax.ShapeDtypeStruct((N, D), jnp.float32),
    )(p0, p1, x, b2)


def _gather_body(h_hbm, idx_hbm, out_hbm, idxv, rows_v, sem):
    cid = lax.axis_index("c")
    sid = lax.axis_index("s")
    wid = cid * NS + sid
    pltpu.sync_copy(idx_hbm.at[wid], idxv)

    def chunk(j, carry):
        pltpu.async_copy(h_hbm.at[idxv.at[j]], rows_v, sem).wait()
        pltpu.sync_copy(rows_v,
                        out_hbm.at[pl.ds(wid * VK * GCH + j * GCH, GCH)])
        return carry

    lax.fori_loop(0, VK, chunk, 0)


_gather_k = pl.kernel(
    _gather_body,
    mesh=_mesh,
    out_type=jax.ShapeDtypeStruct((VPAD, D), jnp.float32),
    scratch_types=[
        pltpu.VMEM((VK, GCH), jnp.int32),
        pltpu.VMEM((GCH, D), jnp.float32),
        pltpu.SemaphoreType.DMA,
    ],
)


def _split_core(arr, pad_val):
    e0 = NS * K0 * CH
    e1 = NS * K1 * CH
    a0 = arr[:e0].reshape(NS, K0, CH)
    a1 = jnp.concatenate(
        [arr[e0:], jnp.full((e1 - (E - e0),), pad_val, jnp.int32)])
    a1 = a1.reshape(NS, K1, CH)
    a1 = jnp.concatenate(
        [a1, jnp.zeros((NS, KMAX - K1, CH), jnp.int32)], axis=1)
    return jnp.concatenate([a0, a1], axis=0)


def _prep_edges(edges):
    col = _split_core(edges[1], 0)
    row = _split_core(edges[0], N)
    return col, row


def kernel(embedding, bias, edges1, edges2, idx_mapping):
    col1, row1 = _prep_edges(edges1)
    col2, row2 = _prep_edges(edges2)
    zeros = jnp.zeros((NROW, D), jnp.float32)
    b2 = bias.reshape(1, D)

    p = _scatter_k(embedding, col1, row1, zeros)
    h1 = _combine(p[0], p[1], embedding, b2)
    p2 = _scatter_k(h1, col2, row2, zeros)
    h2 = _combine(p2[0], p2[1], h1, b2)

    idx = jnp.concatenate([idx_mapping, jnp.zeros((VPAD - V,), jnp.int32)])
    out = _gather_k(h2, idx.reshape(NW, VK, GCH))
    return out[:V]


# speedup vs baseline: 1.7408x; 1.0035x over previous
import jax
import jax.numpy as jnp
from jax import lax
from jax.experimental import pallas as pl
from jax.experimental.pallas import tpu as pltpu
from jax.experimental.pallas import tpu_sc as plsc

NEG_SLOPE = 0.05

N = 10000
D = 128
E = 320000
V = 8000

NC = 2
NS = 16
NW = NC * NS

CH = 128
K0 = 95
K1 = 62
KMAX = K0
RPT = 632
NROW = NS * RPT

VPAD = 8192
GCH = 128
VK = VPAD // (NW * GCH)

_mesh = plsc.VectorSubcoreMesh(core_axis_name="c", subcore_axis_name="s")


def _scatter_body(x_hbm, col_hbm, row_hbm, zeros_hbm, out_hbm,
                  colv, rowv, rows_a, acc, sem_g):
    cid = lax.axis_index("c")
    sid = lax.axis_index("s")
    wid = cid * NS + sid

    pltpu.sync_copy(zeros_hbm.at[pl.ds(sid * RPT, RPT)],
                    acc.at[pl.ds(sid * RPT, RPT)])
    plsc.subcore_barrier()

    pltpu.sync_copy(col_hbm.at[wid], colv)
    pltpu.sync_copy(row_hbm.at[wid], rowv)

    def group(g, carry):
        pltpu.async_copy(x_hbm.at[colv.at[g]], rows_a, sem_g).wait()
        pltpu.sync_copy(rows_a, acc.at[rowv.at[g]], add=True)
        return carry

    kb = jnp.where(cid == 0, K0, K1)
    lax.fori_loop(0, kb, group, 0)
    plsc.subcore_barrier()

    pltpu.sync_copy(acc.at[pl.ds(sid * RPT, RPT)],
                    out_hbm.at[cid, pl.ds(sid * RPT, RPT)])


_scatter_k = pl.kernel(
    _scatter_body,
    mesh=_mesh,
    out_type=jax.ShapeDtypeStruct((NC, NROW, D), jnp.float32),
    scratch_types=[
        pltpu.VMEM((KMAX, CH), jnp.int32),
        pltpu.VMEM((KMAX, CH), jnp.int32),
        pltpu.VMEM((CH, D), jnp.float32),
        pltpu.VMEM_SHARED((NROW, D), jnp.float32),
        pltpu.SemaphoreType.DMA,
    ],
)


def _combine_body(p0_ref, p1_ref, x_ref, b_ref, o_ref):
    s = p0_ref[...] + p1_ref[...] + x_ref[...] + b_ref[...]
    o_ref[...] = jnp.maximum(s, NEG_SLOPE * s)


def _combine(p0, p1, x, b2):
    blk = 400
    return pl.pallas_call(
        _combine_body,
        grid=(N // blk,),
        in_specs=[
            pl.BlockSpec((blk, D), lambda i: (i, 0)),
            pl.BlockSpec((blk, D), lambda i: (i, 0)),
            pl.BlockSpec((blk, D), lambda i: (i, 0)),
            pl.BlockSpec((1, D), lambda i: (0, 0)),
        ],
        out_specs=pl.BlockSpec((blk, D), lambda i: (i, 0)),
        out_shape=jax.ShapeDtypeStruct((N, D), jnp.float32),
    )(p0, p1, x, b2)


def _gather_body(h_hbm, idx_hbm, out_hbm, idxv, rows_v, sem):
    cid = lax.axis_index("c")
    sid = lax.axis_index("s")
    wid = cid * NS + sid
    pltpu.sync_copy(idx_hbm.at[wid], idxv)

    def chunk(j, carry):
        pltpu.async_copy(h_hbm.at[idxv.at[j]], rows_v, sem).wait()
        pltpu.sync_copy(rows_v,
                        out_hbm.at[pl.ds(wid * VK * GCH + j * GCH, GCH)])
        return carry

    lax.fori_loop(0, VK, chunk, 0)


_gather_k = pl.kernel(
    _gather_body,
    mesh=_mesh,
    out_type=jax.ShapeDtypeStruct((VPAD, D), jnp.float32),
    scratch_types=[
        pltpu.VMEM((VK, GCH), jnp.int32),
        pltpu.VMEM((GCH, D), jnp.float32),
        pltpu.SemaphoreType.DMA,
    ],
)


def _split_core(arr, pad_val):
    e0 = NS * K0 * CH
    e1 = NS * K1 * CH
    a0 = arr[:e0].reshape(NS, K0, CH)
    a1 = jnp.concatenate(
        [arr[e0:], jnp.full((e1 - (E - e0),), pad_val, jnp.int32)])
    a1 = a1.reshape(NS, K1, CH)
    a1 = jnp.concatenate(
        [a1, jnp.zeros((NS, KMAX - K1, CH), jnp.int32)], axis=1)
    return jnp.concatenate([a0, a1], axis=0)


def _prep_edges(edges):
    col = _split_core(edges[1], 0)
    row = _split_core(edges[0], N)
    return col, row


def kernel(embedding, bias, edges1, edges2, idx_mapping):
    col1, row1 = _prep_edges(edges1)
    col2, row2 = _prep_edges(edges2)
    zeros = jnp.zeros((NROW, D), jnp.float32)
    b2 = bias.reshape(1, D)

    p = _scatter_k(embedding, col1, row1, zeros)
    h1 = _combine(p[0], p[1], embedding, b2)
    p2 = _scatter_k(h1, col2, row2, zeros)
    h2 = _combine(p2[0], p2[1], h1, b2)

    idx = jnp.concatenate([idx_mapping, jnp.zeros((VPAD - V,), jnp.int32)])
    out = _gather_k(h2, idx.reshape(NW, VK, GCH))
    return out[:V]
